# baseline (speedup 1.0000x reference)
import jax
import jax.numpy as jnp
from jax import lax
from jax.experimental import pallas as pl
from jax.experimental.pallas import tpu as pltpu
from jax.experimental.pallas import tpu_sc as plsc

B = 1024
L = 512
D = 64
N = B * L
VOCAB = 1000000

NC = 2
NS = 16
NW = NC * NS
TPW = N // NW
T = 128
NCHUNK = TPW // T
RPW = TPW // L
LD = L * D


def _emb_body(ids2d_hbm, seg_hbm, tok2_hbm, pos_hbm, segt_hbm, gb_hbm,
              out_hbm, comb_v, ids_v, gidx_v, poff_v, segi_v, rows_v,
              obuf_v, gb_v, sgt_v, sem_i, sem_g0, sem_g1, sem_o):
    wid = lax.axis_index("s") * NC + lax.axis_index("c")

    def ids_pair(c, slot):
        gbase = wid * TPW + c * T
        return (
            pltpu.make_async_copy(
                ids2d_hbm.at[pl.ds(wid * NCHUNK + c, 1)],
                ids_v.at[slot], sem_i),
            pltpu.make_async_copy(
                seg_hbm.at[pl.ds(gbase, T)],
                segi_v.at[slot, pl.ds(0, T)], sem_i),
        )

    def transform_ids(slot):
        for kk in range(8):
            ids = ids_v[slot, 0, pl.ds(16 * kk, 16)]
            gidx_v[slot, pl.ds(16 * kk, 16)] = lax.shift_right_logical(ids, 1)
            poff_v[slot, pl.ds(16 * kk, 16)] = lax.shift_left(ids & 1, 6)

    def gather_cp(c, slot, b):
        sem = sem_g0 if b == 0 else sem_g1
        return pltpu.make_async_copy(
            tok2_hbm.at[gidx_v.at[slot]], rows_v.at[b], sem)

    def out_cps(i, k):
        row = wid * RPW + i
        return [
            pltpu.make_async_copy(
                obuf_v.at[k % 2, pl.ds(td * 8, 8)],
                out_hbm.at[row, td, k], sem_o)
            for td in range(8)]

    def prev_out_cps(i, k):
        if k == 0:
            return out_cps(i - 1, 3)
        return out_cps(i, k - 1)

    for cp in ids_pair(0, 0):
        cp.start()
    for cp in ids_pair(1, 1):
        cp.start()
    for cp in ids_pair(0, 0):
        cp.wait()
    transform_ids(0)
    gather_cp(0, 0, 0).start()

    pltpu.sync_copy(gb_hbm, gb_v)
    pltpu.sync_copy(segt_hbm, sgt_v)
    pltpu.sync_copy(pos_hbm, comb_v.at[pl.ds(0, LD)])
    pltpu.sync_copy(pos_hbm, comb_v.at[pl.ds(LD, LD)])

    s0 = [sgt_v[pl.ds(16 * j, 16)] for j in range(4)]
    s1 = [sgt_v[pl.ds(64 + 16 * j, 16)] for j in range(4)]

    @plsc.parallel_loop(0, L, unroll=4)
    def build(l):
        off = l * D
        for j in range(4):
            o = off + 16 * j
            comb_v[pl.ds(o, 16)] = comb_v[pl.ds(o, 16)] + s0[j]
            o2 = LD + o
            comb_v[pl.ds(o2, 16)] = comb_v[pl.ds(o2, 16)] + s1[j]

    g = [gb_v[pl.ds(16 * j, 16)] for j in range(4)]
    bt = [gb_v[pl.ds(64 + 16 * j, 16)] for j in range(4)]
    iota16 = lax.iota(jnp.int32, 16)

    def do_chunk(i, k):
        c = i * 4 + k
        b = k % 2

        if k == 0:
            @pl.when(c >= 1)
            def _():
                for cp in prev_out_cps(i, k):
                    cp.wait()
        else:
            for cp in prev_out_cps(i, k):
                cp.wait()

        gather_cp(c, k, b).wait()

        def launch_next():
            for cp in ids_pair(c + 1, (k + 1) % 4):
                cp.wait()
            transform_ids((k + 1) % 4)
            gather_cp(c + 1, (k + 1) % 4, (k + 1) % 2).start()

        if k == 3:
            @pl.when(c + 1 < NCHUNK)
            def _():
                launch_next()
        else:
            launch_next()

        def prefetch_ids():
            for cp in ids_pair(c + 2, (k + 2) % 4):
                cp.start()

        if k >= 2:
            @pl.when(c + 2 < NCHUNK)
            def _():
                prefetch_ids()
        else:
            prefetch_ids()

        lb = k * T * D

        @plsc.parallel_loop(0, T, unroll=4)
        def token_body(t):
            sid = segi_v[k, pl.ds(t, 16)][0]
            po = poff_v[k, pl.ds(t, 16)][0]
            coff = sid * LD + lb + t * D
            x = [rows_v[b, t, pl.ds(po + 16 * j, 16)]
                 + comb_v[pl.ds(coff + 16 * j, 16)]
                 for j in range(4)]
            stot = jnp.sum((x[0] + x[1]) + (x[2] + x[3]))
            qtot = jnp.sum((x[0] * x[0] + x[1] * x[1])
                           + (x[2] * x[2] + x[3] * x[3]))
            mean = stot * (1.0 / D)
            var = qtot * (1.0 / D) - mean * mean + 1e-5
            bits = lax.bitcast_convert_type(var, jnp.int32)
            y = lax.bitcast_convert_type(
                jnp.int32(0x5F3759DF) - lax.shift_right_logical(bits, 1),
                jnp.float32)
            for _ in range(3):
                y = y * (1.5 - 0.5 * var * y * y)
            a = y
            nb = mean * a
            for j in range(4):
                rows_v[b, t, pl.ds(po + 16 * j, 16)] = \
                    (x[j] * a - nb) * g[j] + bt[j]

        @plsc.parallel_loop(0, T // 16, unroll=1)
        def trans_body(gg):
            t0 = gg * 16
            tvec = t0 + iota16
            cvec = poff_v[k, pl.ds(t0, 16)]
            src = rows_v.at[b]
            for d in range(D):
                obuf_v[b, d, pl.ds(t0, 16)] = \
                    plsc.load_gather(src, [tvec, cvec])
                if d < D - 1:
                    cvec = cvec + 1

        for cp in out_cps(i, k):
            cp.start()

    def body(i, carry):
        for k in range(4):
            do_chunk(i, k)
        return carry

    lax.fori_loop(0, NCHUNK // 4, body, 0)
    for cp in out_cps(NCHUNK // 4 - 1, 3):
        cp.wait()


@jax.jit
def _emb_call(ids2d, seg_flat, tok2, pos_flat, segt_flat, gb):
    mesh = plsc.VectorSubcoreMesh(core_axis_name="c", subcore_axis_name="s")
    f = pl.kernel(
        _emb_body,
        out_type=jax.ShapeDtypeStruct((B, 8, 4, 8, 128), jnp.float32),
        mesh=mesh,
        compiler_params=pltpu.CompilerParams(needs_layout_passes=False,
                                             use_tc_tiling_on_sc=False),
        scratch_types=[
            pltpu.VMEM((2 * LD,), jnp.float32),
            pltpu.VMEM((4, 1, 128), jnp.int32),
            pltpu.VMEM((4, 128), jnp.int32),
            pltpu.VMEM((4, T + 16), jnp.int32),
            pltpu.VMEM((4, T + 16), jnp.int32),
            pltpu.VMEM((2, T, 128), jnp.float32),
            pltpu.VMEM((2, D, T), jnp.float32),
            pltpu.VMEM((2 * D,), jnp.float32),
            pltpu.VMEM((2 * D,), jnp.float32),
            pltpu.SemaphoreType.DMA,
            pltpu.SemaphoreType.DMA,
            pltpu.SemaphoreType.DMA,
            pltpu.SemaphoreType.DMA,
        ],
    )
    return f(ids2d, seg_flat, tok2, pos_flat, segt_flat, gb)


def kernel(input_ids, segment_ids, tok_table, pos_table, seg_table, gamma, beta):
    ids2d = input_ids.astype(jnp.int32).reshape(N // 128, 128)
    seg_flat = segment_ids.astype(jnp.int32).reshape(N)
    tok2 = tok_table.reshape(VOCAB // 2, 2 * D)
    pos_flat = pos_table.reshape(LD)
    segt_flat = seg_table.reshape(2 * D)
    gb = jnp.concatenate([gamma, beta]).astype(jnp.float32)
    out5 = _emb_call(ids2d, seg_flat, tok2, pos_flat, segt_flat, gb)
    return out5.transpose(0, 2, 4, 1, 3).reshape(B, L, D)


# speedup vs baseline: 1.5669x; 1.5669x over previous
import jax
import jax.numpy as jnp
from jax import lax
from jax.experimental import pallas as pl
from jax.experimental.pallas import tpu as pltpu
from jax.experimental.pallas import tpu_sc as plsc

B = 1024
L = 512
D = 64
N = B * L
VOCAB = 1000000

NC = 2
NS = 16
NW = NC * NS
TPW = N // NW
T = 128
NCHUNK = TPW // T
RPW = TPW // L
LD = L * D


def _emb_body(ids2d_hbm, seg_hbm, tok2_hbm, pos_hbm, segt_hbm, gb_hbm,
              out_hbm, comb_v, ids_v, gidx_v, poff_v, segi_v, rows_v,
              obuf_v, gb_v, sgt_v, sem_i, sem_g0, sem_g1, sem_o):
    wid = lax.axis_index("s") * NC + lax.axis_index("c")

    def ids_pair(c, slot):
        gbase = wid * TPW + c * T
        return (
            pltpu.make_async_copy(
                ids2d_hbm.at[pl.ds(wid * NCHUNK + c, 1)],
                ids_v.at[slot], sem_i),
            pltpu.make_async_copy(
                seg_hbm.at[pl.ds(gbase, T)],
                segi_v.at[slot, pl.ds(0, T)], sem_i),
        )

    def transform_ids(slot):
        for kk in range(8):
            ids = ids_v[slot, 0, pl.ds(16 * kk, 16)]
            gidx_v[slot, pl.ds(16 * kk, 16)] = lax.shift_right_logical(ids, 1)
            poff_v[slot, pl.ds(16 * kk, 16)] = lax.shift_left(ids & 1, 6)

    def gather_cp(c, slot, b):
        sem = sem_g0 if b == 0 else sem_g1
        return pltpu.make_async_copy(
            tok2_hbm.at[gidx_v.at[slot]], rows_v.at[b], sem)

    def out_cps(i, k):
        row = wid * RPW + i
        return [
            pltpu.make_async_copy(
                obuf_v.at[k % 2, pl.ds(td * 8, 8), pl.ds(0, T)],
                out_hbm.at[row, td, k], sem_o)
            for td in range(8)]

    def prev_out_cps(i, k):
        if k == 0:
            return out_cps(i - 1, 3)
        return out_cps(i, k - 1)

    for cp in ids_pair(0, 0):
        cp.start()
    for cp in ids_pair(1, 1):
        cp.start()
    for cp in ids_pair(0, 0):
        cp.wait()
    transform_ids(0)
    gather_cp(0, 0, 0).start()

    pltpu.sync_copy(gb_hbm, gb_v)
    pltpu.sync_copy(segt_hbm, sgt_v)
    pltpu.sync_copy(pos_hbm, comb_v.at[pl.ds(0, LD)])
    pltpu.sync_copy(pos_hbm, comb_v.at[pl.ds(LD, LD)])

    s0 = [sgt_v[pl.ds(16 * j, 16)] for j in range(4)]
    s1 = [sgt_v[pl.ds(64 + 16 * j, 16)] for j in range(4)]

    @plsc.parallel_loop(0, L, unroll=4)
    def build(l):
        off = l * D
        for j in range(4):
            o = off + 16 * j
            comb_v[pl.ds(o, 16)] = comb_v[pl.ds(o, 16)] + s0[j]
            o2 = LD + o
            comb_v[pl.ds(o2, 16)] = comb_v[pl.ds(o2, 16)] + s1[j]

    g = [gb_v[pl.ds(16 * j, 16)] for j in range(4)]
    bt = [gb_v[pl.ds(64 + 16 * j, 16)] for j in range(4)]
    iota16 = lax.iota(jnp.int32, 16)
    iod = [16 * j + iota16 for j in range(4)]

    def do_chunk(i, k):
        c = i * 4 + k
        b = k % 2

        if k == 0:
            @pl.when(c >= 1)
            def _():
                for cp in prev_out_cps(i, k):
                    cp.wait()
        else:
            for cp in prev_out_cps(i, k):
                cp.wait()

        gather_cp(c, k, b).wait()

        def launch_next():
            for cp in ids_pair(c + 1, (k + 1) % 4):
                cp.wait()
            transform_ids((k + 1) % 4)
            gather_cp(c + 1, (k + 1) % 4, (k + 1) % 2).start()

        if k == 3:
            @pl.when(c + 1 < NCHUNK)
            def _():
                launch_next()
        else:
            launch_next()

        def prefetch_ids():
            for cp in ids_pair(c + 2, (k + 2) % 4):
                cp.start()

        if k >= 2:
            @pl.when(c + 2 < NCHUNK)
            def _():
                prefetch_ids()
        else:
            prefetch_ids()

        lb = k * T * D

        @plsc.parallel_loop(0, T, unroll=4)
        def token_body(t):
            sid = segi_v[k, pl.ds(t, 16)][0]
            po = poff_v[k, pl.ds(t, 16)][0]
            coff = sid * LD + lb + t * D
            x = [rows_v[b, t, pl.ds(po + 16 * j, 16)]
                 + comb_v[pl.ds(coff + 16 * j, 16)]
                 for j in range(4)]
            stot = jnp.sum((x[0] + x[1]) + (x[2] + x[3]))
            qtot = jnp.sum((x[0] * x[0] + x[1] * x[1])
                           + (x[2] * x[2] + x[3] * x[3]))
            mean = stot * (1.0 / D)
            var = qtot * (1.0 / D) - mean * mean + 1e-5
            bits = lax.bitcast_convert_type(var, jnp.int32)
            y = lax.bitcast_convert_type(
                jnp.int32(0x5F3759DF) - lax.shift_right_logical(bits, 1),
                jnp.float32)
            for _ in range(3):
                y = y * (1.5 - 0.5 * var * y * y)
            a = y
            nb = mean * a
            ob = obuf_v.at[b]
            tvec = jnp.broadcast_to(t, (16,))
            for j in range(4):
                plsc.store_scatter(ob, [iod[j], tvec],
                                   (x[j] * a - nb) * g[j] + bt[j])

        for cp in out_cps(i, k):
            cp.start()

    def body(i, carry):
        for k in range(4):
            do_chunk(i, k)
        return carry

    lax.fori_loop(0, NCHUNK // 4, body, 0)
    for cp in out_cps(NCHUNK // 4 - 1, 3):
        cp.wait()


@jax.jit
def _emb_call(ids2d, seg_flat, tok2, pos_flat, segt_flat, gb):
    mesh = plsc.VectorSubcoreMesh(core_axis_name="c", subcore_axis_name="s")
    f = pl.kernel(
        _emb_body,
        out_type=jax.ShapeDtypeStruct((B, 8, 4, 8, 128), jnp.float32),
        mesh=mesh,
        compiler_params=pltpu.CompilerParams(needs_layout_passes=False,
                                             use_tc_tiling_on_sc=False),
        scratch_types=[
            pltpu.VMEM((2 * LD,), jnp.float32),
            pltpu.VMEM((4, 1, 128), jnp.int32),
            pltpu.VMEM((4, 128), jnp.int32),
            pltpu.VMEM((4, T + 16), jnp.int32),
            pltpu.VMEM((4, T + 16), jnp.int32),
            pltpu.VMEM((2, T, 128), jnp.float32),
            pltpu.VMEM((2, D, T + 1), jnp.float32),
            pltpu.VMEM((2 * D,), jnp.float32),
            pltpu.VMEM((2 * D,), jnp.float32),
            pltpu.SemaphoreType.DMA,
            pltpu.SemaphoreType.DMA,
            pltpu.SemaphoreType.DMA,
            pltpu.SemaphoreType.DMA,
        ],
    )
    return f(ids2d, seg_flat, tok2, pos_flat, segt_flat, gb)


def kernel(input_ids, segment_ids, tok_table, pos_table, seg_table, gamma, beta):
    ids2d = input_ids.astype(jnp.int32).reshape(N // 128, 128)
    seg_flat = segment_ids.astype(jnp.int32).reshape(N)
    tok2 = tok_table.reshape(VOCAB // 2, 2 * D)
    pos_flat = pos_table.reshape(LD)
    segt_flat = seg_table.reshape(2 * D)
    gb = jnp.concatenate([gamma, beta]).astype(jnp.float32)
    out5 = _emb_call(ids2d, seg_flat, tok2, pos_flat, segt_flat, gb)
    return out5.transpose(0, 2, 4, 1, 3).reshape(B, L, D)


# speedup vs baseline: 1.7941x; 1.1451x over previous
import jax
import jax.numpy as jnp
from jax import lax
from jax.experimental import pallas as pl
from jax.experimental.pallas import tpu as pltpu
from jax.experimental.pallas import tpu_sc as plsc

B = 1024
L = 512
D = 64
N = B * L
VOCAB = 1000000

NC = 2
NS = 16
NW = NC * NS
TPW = N // NW
T = 128
NCHUNK = TPW // T
RPW = TPW // L
LD = L * D


def _emb_body(ids2d_hbm, seg_hbm, tok2_hbm, pos_hbm, segt_hbm, gb_hbm,
              out_hbm, comb_v, ids_v, segi_v, rows_v,
              obuf_v, gb_v, sgt_v, sem_i, sem_g0, sem_g1, sem_o):
    wid = lax.axis_index("s") * NC + lax.axis_index("c")

    def ids_pair(c, slot):
        gbase = wid * TPW + c * T
        return (
            pltpu.make_async_copy(
                ids2d_hbm.at[pl.ds(wid * NCHUNK + c, 1)],
                ids_v.at[slot], sem_i),
            pltpu.make_async_copy(
                seg_hbm.at[pl.ds(gbase, T)],
                segi_v.at[slot, pl.ds(0, T)], sem_i),
        )

    def gather_cp(c, slot, b):
        sem = sem_g0 if b == 0 else sem_g1
        return pltpu.make_async_copy(
            tok2_hbm.at[ids_v.at[slot, 0]], rows_v.at[b], sem)

    def out_cps(i, k):
        row = wid * RPW + i
        return [
            pltpu.make_async_copy(
                obuf_v.at[k % 2, pl.ds(td * 8, 8), pl.ds(0, T)],
                out_hbm.at[row, td, k], sem_o)
            for td in range(8)]

    def prev_out_cps(i, k):
        if k == 0:
            return out_cps(i - 1, 3)
        return out_cps(i, k - 1)

    for cp in ids_pair(0, 0):
        cp.start()
    for cp in ids_pair(1, 1):
        cp.start()
    for cp in ids_pair(0, 0):
        cp.wait()
    gather_cp(0, 0, 0).start()

    pltpu.sync_copy(gb_hbm, gb_v)
    pltpu.sync_copy(segt_hbm, sgt_v)
    pltpu.sync_copy(pos_hbm, comb_v.at[pl.ds(0, LD)])
    pltpu.sync_copy(pos_hbm, comb_v.at[pl.ds(LD, LD)])

    s0 = [sgt_v[pl.ds(16 * j, 16)] for j in range(4)]
    s1 = [sgt_v[pl.ds(64 + 16 * j, 16)] for j in range(4)]

    @plsc.parallel_loop(0, L, unroll=4)
    def build(l):
        off = l * D
        for j in range(4):
            o = off + 16 * j
            comb_v[pl.ds(o, 16)] = comb_v[pl.ds(o, 16)] + s0[j]
            o2 = LD + o
            comb_v[pl.ds(o2, 16)] = comb_v[pl.ds(o2, 16)] + s1[j]

    g = [gb_v[pl.ds(16 * j, 16)] for j in range(4)]
    bt = [gb_v[pl.ds(64 + 16 * j, 16)] for j in range(4)]
    iota16 = lax.iota(jnp.int32, 16)
    iod = [16 * j + iota16 for j in range(4)]

    def do_chunk(i, k):
        c = i * 4 + k
        b = k % 2

        if k == 0:
            @pl.when(c >= 1)
            def _():
                for cp in prev_out_cps(i, k):
                    cp.wait()
        else:
            for cp in prev_out_cps(i, k):
                cp.wait()

        gather_cp(c, k, b).wait()

        def launch_next():
            for cp in ids_pair(c + 1, (k + 1) % 4):
                cp.wait()
            gather_cp(c + 1, (k + 1) % 4, (k + 1) % 2).start()

        if k == 3:
            @pl.when(c + 1 < NCHUNK)
            def _():
                launch_next()
        else:
            launch_next()

        def prefetch_ids():
            for cp in ids_pair(c + 2, (k + 2) % 4):
                cp.start()

        if k >= 2:
            @pl.when(c + 2 < NCHUNK)
            def _():
                prefetch_ids()
        else:
            prefetch_ids()

        lb = k * T * D

        @plsc.parallel_loop(0, T, unroll=4)
        def token_body(t):
            sid = segi_v[k, pl.ds(t, 16)][0]
            coff = sid * LD + lb + t * D
            x = [rows_v[b, t, pl.ds(16 * j, 16)]
                 + comb_v[pl.ds(coff + 16 * j, 16)]
                 for j in range(4)]
            stot = jnp.sum((x[0] + x[1]) + (x[2] + x[3]))
            qtot = jnp.sum((x[0] * x[0] + x[1] * x[1])
                           + (x[2] * x[2] + x[3] * x[3]))
            mean = stot * (1.0 / D)
            var = qtot * (1.0 / D) - mean * mean + 1e-5
            bits = lax.bitcast_convert_type(var, jnp.int32)
            y = lax.bitcast_convert_type(
                jnp.int32(0x5F3759DF) - lax.shift_right_logical(bits, 1),
                jnp.float32)
            for _ in range(3):
                y = y * (1.5 - 0.5 * var * y * y)
            a = y
            nb = mean * a
            ob = obuf_v.at[b]
            tvec = jnp.broadcast_to(t, (16,))
            for j in range(4):
                plsc.store_scatter(ob, [iod[j], tvec],
                                   (x[j] * a - nb) * g[j] + bt[j])

        for cp in out_cps(i, k):
            cp.start()

    def body(i, carry):
        for k in range(4):
            do_chunk(i, k)
        return carry

    lax.fori_loop(0, NCHUNK // 4, body, 0)
    for cp in out_cps(NCHUNK // 4 - 1, 3):
        cp.wait()


@jax.jit
def _emb_call(ids2d, seg_flat, tok2, pos_flat, segt_flat, gb):
    mesh = plsc.VectorSubcoreMesh(core_axis_name="c", subcore_axis_name="s")
    f = pl.kernel(
        _emb_body,
        out_type=jax.ShapeDtypeStruct((B, 8, 4, 8, 128), jnp.float32),
        mesh=mesh,
        compiler_params=pltpu.CompilerParams(needs_layout_passes=False,
                                             use_tc_tiling_on_sc=False),
        scratch_types=[
            pltpu.VMEM((2 * LD,), jnp.float32),
            pltpu.VMEM((4, 1, 128), jnp.int32),
            pltpu.VMEM((4, T + 16), jnp.int32),
            pltpu.VMEM((2, T, 128), jnp.float32),
            pltpu.VMEM((2, D, T + 1), jnp.float32),
            pltpu.VMEM((2 * D,), jnp.float32),
            pltpu.VMEM((2 * D,), jnp.float32),
            pltpu.SemaphoreType.DMA,
            pltpu.SemaphoreType.DMA,
            pltpu.SemaphoreType.DMA,
            pltpu.SemaphoreType.DMA,
        ],
    )
    return f(ids2d, seg_flat, tok2, pos_flat, segt_flat, gb)


def kernel(input_ids, segment_ids, tok_table, pos_table, seg_table, gamma, beta):
    ids2d = input_ids.astype(jnp.int32).reshape(N // 128, 128)
    seg_flat = segment_ids.astype(jnp.int32).reshape(N)
    tok2 = jnp.pad(tok_table, ((0, 0), (0, D)))
    pos_flat = pos_table.reshape(LD)
    segt_flat = seg_table.reshape(2 * D)
    gb = jnp.concatenate([gamma, beta]).astype(jnp.float32)
    out5 = _emb_call(ids2d, seg_flat, tok2, pos_flat, segt_flat, gb)
    return out5.transpose(0, 2, 4, 1, 3).reshape(B, L, D)
